# Initial kernel scaffold; baseline (speedup 1.0000x reference)
import functools

import jax
import jax.numpy as jnp
from jax import lax
from jax.experimental import pallas as pl
from jax.experimental.pallas import tpu as pltpu
from jax.experimental.pallas import tpu_sc as plsc

N = 10000
E = 320000
D_IN = 128
D_H = 128
D_Z = 64

NC, NS = 2, 16
NT = NC * NS
EPT = E // NT
K = 80
CH = EPT // K
RPT = N // NS
ZR = 125

_MESH = plsc.VectorSubcoreMesh(core_axis_name="c", subcore_axis_name="s")


def _make_seg(D):
    def body(hn, srcs, dsts, out, sidx, didx, rows, zbuf, accum, sem):
        c = lax.axis_index("c")
        s = lax.axis_index("s")
        base = (c * NS + s) * EPT

        nlanes = D // 16

        def zb(i, _):
            zbuf[i // nlanes, pl.ds((i % nlanes) * 16, 16)] = jnp.zeros(
                (16,), jnp.float32)
            return 0

        lax.fori_loop(0, ZR * nlanes, zb, 0)
        for r in range(RPT // ZR):
            pltpu.sync_copy(zbuf, accum.at[pl.ds(s * RPT + r * ZR, ZR)])
        plsc.subcore_barrier()

        def chunk(j, _):
            off = base + j * K
            pltpu.sync_copy(srcs.at[pl.ds(off, K)], sidx)
            pltpu.sync_copy(dsts.at[pl.ds(off, K)], didx)
            pltpu.async_copy(hn.at[sidx], rows, sem).wait()
            pltpu.sync_copy(rows, accum.at[didx], add=True)
            return 0

        lax.fori_loop(0, CH, chunk, 0)
        plsc.subcore_barrier()
        pltpu.sync_copy(accum.at[pl.ds(s * RPT, RPT)],
                        out.at[c].at[pl.ds(s * RPT, RPT)])

    return pl.kernel(
        body,
        out_type=jax.ShapeDtypeStruct((NC, N, D), jnp.float32),
        mesh=_MESH,
        scratch_types=[
            pltpu.VMEM((K,), jnp.int32),
            pltpu.VMEM((K,), jnp.int32),
            pltpu.VMEM((K, D), jnp.float32),
            pltpu.VMEM((ZR, D), jnp.float32),
            pltpu.VMEM_SHARED((N, D), jnp.float32),
            pltpu.SemaphoreType.DMA,
        ],
    )


def _make_deg():
    DD = 16

    def body(dsts, out, didx, ones, zbuf, accum, sem):
        c = lax.axis_index("c")
        s = lax.axis_index("s")
        base = (c * NS + s) * EPT

        def fill_ones(i, _):
            ones[i, pl.ds(0, 16)] = jnp.ones((16,), jnp.float32)
            return 0

        lax.fori_loop(0, K, fill_ones, 0)

        def zb(i, _):
            zbuf[i, pl.ds(0, 16)] = jnp.zeros((16,), jnp.float32)
            return 0

        lax.fori_loop(0, ZR, zb, 0)
        for r in range(RPT // ZR):
            pltpu.sync_copy(zbuf, accum.at[pl.ds(s * RPT + r * ZR, ZR)])
        plsc.subcore_barrier()

        def chunk(j, _):
            off = base + j * K
            pltpu.sync_copy(dsts.at[pl.ds(off, K)], didx)
            pltpu.sync_copy(ones, accum.at[didx], add=True)
            return 0

        lax.fori_loop(0, CH, chunk, 0)
        plsc.subcore_barrier()
        pltpu.sync_copy(accum.at[pl.ds(s * RPT, RPT)],
                        out.at[c].at[pl.ds(s * RPT, RPT)])

    return pl.kernel(
        body,
        out_type=jax.ShapeDtypeStruct((NC, N, DD), jnp.float32),
        mesh=_MESH,
        scratch_types=[
            pltpu.VMEM((K,), jnp.int32),
            pltpu.VMEM((K, DD), jnp.float32),
            pltpu.VMEM((ZR, DD), jnp.float32),
            pltpu.VMEM_SHARED((N, DD), jnp.float32),
            pltpu.SemaphoreType.DMA,
        ],
    )


BM = 1000
GR = N // BM


def _dinv_block(degp):
    deg = degp[0, :, 0] + degp[1, :, 0] + 1.0
    return lax.rsqrt(deg)


def _make_mm_scale(Dout):
    def body(x_ref, w_ref, degp_ref, hn_ref):
        dinv = _dinv_block(degp_ref[...])
        hn_ref[...] = dinv[:, None] * jnp.dot(
            x_ref[...], w_ref[...], preferred_element_type=jnp.float32)

    return pl.pallas_call(
        body,
        grid=(GR,),
        in_specs=[
            pl.BlockSpec((BM, D_H), lambda i: (i, 0)),
            pl.BlockSpec((D_H, Dout), lambda i: (0, 0)),
            pl.BlockSpec((2, BM, 16), lambda i: (0, i, 0)),
        ],
        out_specs=pl.BlockSpec((BM, Dout), lambda i: (i, 0)),
        out_shape=jax.ShapeDtypeStruct((N, Dout), jnp.float32),
    )


def _make_combine_stats(D):
    def body(p_ref, hn_ref, degp_ref, b_ref, y_ref, st_ref):
        i = pl.program_id(0)
        dinv = _dinv_block(degp_ref[...])
        y = dinv[:, None] * (p_ref[0] + p_ref[1] + hn_ref[...]) + b_ref[...]
        y_ref[...] = y
        su = jnp.sum(y, axis=0)
        sq = jnp.sum(y * y, axis=0)
        upd = jnp.zeros((8, D), jnp.float32).at[0].set(su).at[1].set(sq)

        @pl.when(i == 0)
        def _():
            st_ref[...] = upd

        @pl.when(i > 0)
        def _():
            st_ref[...] += upd

    return pl.pallas_call(
        body,
        grid=(GR,),
        in_specs=[
            pl.BlockSpec((2, BM, D), lambda i: (0, i, 0)),
            pl.BlockSpec((BM, D), lambda i: (i, 0)),
            pl.BlockSpec((2, BM, 16), lambda i: (0, i, 0)),
            pl.BlockSpec((1, D), lambda i: (0, 0)),
        ],
        out_specs=[
            pl.BlockSpec((BM, D), lambda i: (i, 0)),
            pl.BlockSpec((8, D), lambda i: (0, 0)),
        ],
        out_shape=[
            jax.ShapeDtypeStruct((N, D), jnp.float32),
            jax.ShapeDtypeStruct((8, D), jnp.float32),
        ],
    )


def _make_bn_act_mm(D, Dout):
    def body(y_ref, st_ref, degp_ref, g_ref, bt_ref, w_ref, hn_ref):
        dinv = _dinv_block(degp_ref[...])
        st = st_ref[...]
        m = st[0:1, :] / N
        v = st[1:2, :] / N - m * m
        a = (y_ref[...] - m) * lax.rsqrt(v + 1e-5) * g_ref[...] + bt_ref[...]
        a = jnp.where(a >= 0, a, 0.01 * a)
        hn_ref[...] = dinv[:, None] * jnp.dot(
            a, w_ref[...], preferred_element_type=jnp.float32)

    return pl.pallas_call(
        body,
        grid=(GR,),
        in_specs=[
            pl.BlockSpec((BM, D), lambda i: (i, 0)),
            pl.BlockSpec((8, D), lambda i: (0, 0)),
            pl.BlockSpec((2, BM, 16), lambda i: (0, i, 0)),
            pl.BlockSpec((1, D), lambda i: (0, 0)),
            pl.BlockSpec((1, D), lambda i: (0, 0)),
            pl.BlockSpec((D, Dout), lambda i: (0, 0)),
        ],
        out_specs=pl.BlockSpec((BM, Dout), lambda i: (i, 0)),
        out_shape=jax.ShapeDtypeStruct((N, Dout), jnp.float32),
    )


def _make_final(D):
    def body(p_ref, hn_ref, degp_ref, b_ref, out_ref):
        dinv = _dinv_block(degp_ref[...])
        out_ref[...] = dinv[:, None] * (
            p_ref[0] + p_ref[1] + hn_ref[...]) + b_ref[...]

    return pl.pallas_call(
        body,
        grid=(GR,),
        in_specs=[
            pl.BlockSpec((2, BM, D), lambda i: (0, i, 0)),
            pl.BlockSpec((BM, D), lambda i: (i, 0)),
            pl.BlockSpec((2, BM, 16), lambda i: (0, i, 0)),
            pl.BlockSpec((1, D), lambda i: (0, 0)),
        ],
        out_specs=pl.BlockSpec((BM, D), lambda i: (i, 0)),
        out_shape=jax.ShapeDtypeStruct((N, D), jnp.float32),
    )


_deg = _make_deg()
_seg_h = _make_seg(D_H)
_seg_z = _make_seg(D_Z)
_mm_scale = _make_mm_scale(D_H)
_combine_stats = _make_combine_stats(D_H)
_bn_act_mm_h = _make_bn_act_mm(D_H, D_H)
_bn_act_mm_z = _make_bn_act_mm(D_H, D_Z)
_final = _make_final(D_Z)


@jax.jit
def kernel(x, edge_index, W1, b1, g1, bt1, W2, b2, g2, bt2,
           W3, b3, g3, bt3, W4, b4):
    src = edge_index[0]
    dst = edge_index[1]
    degp = _deg(dst)

    hn = _mm_scale(x, W1, degp)
    p = _seg_h(hn, src, dst)
    y, st = _combine_stats(p, hn, degp, b1.reshape(1, -1))

    hn = _bn_act_mm_h(y, st, degp, g1.reshape(1, -1), bt1.reshape(1, -1), W2)
    p = _seg_h(hn, src, dst)
    y, st = _combine_stats(p, hn, degp, b2.reshape(1, -1))

    hn = _bn_act_mm_h(y, st, degp, g2.reshape(1, -1), bt2.reshape(1, -1), W3)
    p = _seg_h(hn, src, dst)
    y, st = _combine_stats(p, hn, degp, b3.reshape(1, -1))

    hn = _bn_act_mm_z(y, st, degp, g3.reshape(1, -1), bt3.reshape(1, -1), W4)
    p = _seg_z(hn, src, dst)
    return _final(p, hn, degp, b4.reshape(1, -1))


# speedup vs baseline: 9.5882x; 9.5882x over previous
import functools

import jax
import jax.numpy as jnp
from jax import lax
from jax.experimental import pallas as pl
from jax.experimental.pallas import tpu as pltpu
from jax.experimental.pallas import tpu_sc as plsc

N = 10000
E = 320000
D_IN = 128
D_H = 128
D_Z = 64

NC, NS = 2, 16
NT = NC * NS
EPT = E // NT
K = 80
CH = EPT // K
N2 = 10240
RPT = N2 // NS
ZR = 128

_MESH = plsc.VectorSubcoreMesh(core_axis_name="c", subcore_axis_name="s")


def _make_seg(D):
    def body(hn, srcs, dsts, out, sidx, didx, rows, zbuf, accum, sem):
        c = lax.axis_index("c")
        s = lax.axis_index("s")
        base = (c * NS + s) * EPT

        nlanes = D // 16

        def zb(i, _):
            zbuf[i // nlanes, pl.ds((i % nlanes) * 16, 16)] = jnp.zeros(
                (16,), jnp.float32)
            return 0

        lax.fori_loop(0, ZR * nlanes, zb, 0)
        for r in range(RPT // ZR):
            pltpu.sync_copy(zbuf, accum.at[pl.ds(s * RPT + r * ZR, ZR)])
        plsc.subcore_barrier()

        def chunk(j, _):
            off = base + j * K
            pltpu.sync_copy(srcs.at[pl.ds(off, K)], sidx)
            pltpu.sync_copy(dsts.at[pl.ds(off, K)], didx)
            pltpu.async_copy(hn.at[sidx], rows, sem).wait()
            pltpu.sync_copy(rows, accum.at[didx], add=True)
            return 0

        lax.fori_loop(0, CH, chunk, 0)
        plsc.subcore_barrier()
        pltpu.sync_copy(accum.at[pl.ds(s * RPT, RPT)],
                        out.at[c].at[pl.ds(s * RPT, RPT)])

    return pl.kernel(
        body,
        out_type=jax.ShapeDtypeStruct((NC, N2, D), jnp.float32),
        mesh=_MESH,
        scratch_types=[
            pltpu.VMEM((K,), jnp.int32),
            pltpu.VMEM((K,), jnp.int32),
            pltpu.VMEM((K, D), jnp.float32),
            pltpu.VMEM((ZR, D), jnp.float32),
            pltpu.VMEM_SHARED((N2, D), jnp.float32),
            pltpu.SemaphoreType.DMA,
        ],
    )


def _make_deg():
    DD = 128

    def body(dsts, out, didx, ones, zbuf, accum, sem):
        c = lax.axis_index("c")
        s = lax.axis_index("s")
        base = (c * NS + s) * EPT
        nlanes = DD // 16

        def fill_ones(i, _):
            ones[i // nlanes, pl.ds((i % nlanes) * 16, 16)] = jnp.ones(
                (16,), jnp.float32)
            return 0

        lax.fori_loop(0, K * nlanes, fill_ones, 0)

        def zb(i, _):
            zbuf[i // nlanes, pl.ds((i % nlanes) * 16, 16)] = jnp.zeros(
                (16,), jnp.float32)
            return 0

        lax.fori_loop(0, ZR * nlanes, zb, 0)
        for r in range(RPT // ZR):
            pltpu.sync_copy(zbuf, accum.at[pl.ds(s * RPT + r * ZR, ZR)])
        plsc.subcore_barrier()

        def chunk(j, _):
            off = base + j * K
            pltpu.sync_copy(dsts.at[pl.ds(off, K)], didx)
            pltpu.sync_copy(ones, accum.at[didx], add=True)
            return 0

        lax.fori_loop(0, CH, chunk, 0)
        plsc.subcore_barrier()
        pltpu.sync_copy(accum.at[pl.ds(s * RPT, RPT)],
                        out.at[c].at[pl.ds(s * RPT, RPT)])

    return pl.kernel(
        body,
        out_type=jax.ShapeDtypeStruct((NC, N2, DD), jnp.float32),
        mesh=_MESH,
        scratch_types=[
            pltpu.VMEM((K,), jnp.int32),
            pltpu.VMEM((K, DD), jnp.float32),
            pltpu.VMEM((ZR, DD), jnp.float32),
            pltpu.VMEM_SHARED((N2, DD), jnp.float32),
            pltpu.SemaphoreType.DMA,
        ],
    )


BM = 1000
GR = N // BM


def _dinv_block(degp):
    deg = degp[0, :, 0] + degp[1, :, 0] + 1.0
    return lax.rsqrt(deg)


def _make_mm_scale(Dout):
    def body(x_ref, w_ref, degp_ref, hn_ref):
        dinv = _dinv_block(degp_ref[...])
        hn_ref[...] = dinv[:, None] * jnp.dot(
            x_ref[...], w_ref[...], preferred_element_type=jnp.float32)

    return pl.pallas_call(
        body,
        grid=(GR,),
        in_specs=[
            pl.BlockSpec((BM, D_H), lambda i: (i, 0)),
            pl.BlockSpec((D_H, Dout), lambda i: (0, 0)),
            pl.BlockSpec((2, BM, 128), lambda i: (0, i, 0)),
        ],
        out_specs=pl.BlockSpec((BM, Dout), lambda i: (i, 0)),
        out_shape=jax.ShapeDtypeStruct((N, Dout), jnp.float32),
    )


def _make_combine_stats(D):
    def body(p_ref, hn_ref, degp_ref, b_ref, y_ref, st_ref):
        i = pl.program_id(0)
        dinv = _dinv_block(degp_ref[...])
        y = dinv[:, None] * (p_ref[0] + p_ref[1] + hn_ref[...]) + b_ref[...]
        y_ref[...] = y
        su = jnp.sum(y, axis=0)
        sq = jnp.sum(y * y, axis=0)
        upd = jnp.concatenate(
            [su.reshape(1, D), sq.reshape(1, D),
             jnp.zeros((6, D), jnp.float32)], axis=0)

        @pl.when(i == 0)
        def _():
            st_ref[...] = upd

        @pl.when(i > 0)
        def _():
            st_ref[...] += upd

    return pl.pallas_call(
        body,
        grid=(GR,),
        in_specs=[
            pl.BlockSpec((2, BM, D), lambda i: (0, i, 0)),
            pl.BlockSpec((BM, D), lambda i: (i, 0)),
            pl.BlockSpec((2, BM, 128), lambda i: (0, i, 0)),
            pl.BlockSpec((1, D), lambda i: (0, 0)),
        ],
        out_specs=[
            pl.BlockSpec((BM, D), lambda i: (i, 0)),
            pl.BlockSpec((8, D), lambda i: (0, 0)),
        ],
        out_shape=[
            jax.ShapeDtypeStruct((N, D), jnp.float32),
            jax.ShapeDtypeStruct((8, D), jnp.float32),
        ],
    )


def _make_bn_act_mm(D, Dout):
    def body(y_ref, st_ref, degp_ref, g_ref, bt_ref, w_ref, hn_ref):
        dinv = _dinv_block(degp_ref[...])
        st = st_ref[...]
        m = st[0:1, :] / N
        v = st[1:2, :] / N - m * m
        a = (y_ref[...] - m) * lax.rsqrt(v + 1e-5) * g_ref[...] + bt_ref[...]
        a = jnp.where(a >= 0, a, 0.01 * a)
        hn_ref[...] = dinv[:, None] * jnp.dot(
            a, w_ref[...], preferred_element_type=jnp.float32)

    return pl.pallas_call(
        body,
        grid=(GR,),
        in_specs=[
            pl.BlockSpec((BM, D), lambda i: (i, 0)),
            pl.BlockSpec((8, D), lambda i: (0, 0)),
            pl.BlockSpec((2, BM, 128), lambda i: (0, i, 0)),
            pl.BlockSpec((1, D), lambda i: (0, 0)),
            pl.BlockSpec((1, D), lambda i: (0, 0)),
            pl.BlockSpec((D, Dout), lambda i: (0, 0)),
        ],
        out_specs=pl.BlockSpec((BM, Dout), lambda i: (i, 0)),
        out_shape=jax.ShapeDtypeStruct((N, Dout), jnp.float32),
    )


def _make_final(D, Dout):
    def body(p_ref, hn_ref, degp_ref, b_ref, out_ref):
        dinv = _dinv_block(degp_ref[...])
        full = dinv[:, None] * (p_ref[0] + p_ref[1] + hn_ref[...])
        out_ref[...] = full[:, :Dout] + b_ref[...]

    return pl.pallas_call(
        body,
        grid=(GR,),
        in_specs=[
            pl.BlockSpec((2, BM, D), lambda i: (0, i, 0)),
            pl.BlockSpec((BM, D), lambda i: (i, 0)),
            pl.BlockSpec((2, BM, 128), lambda i: (0, i, 0)),
            pl.BlockSpec((1, Dout), lambda i: (0, 0)),
        ],
        out_specs=pl.BlockSpec((BM, Dout), lambda i: (i, 0)),
        out_shape=jax.ShapeDtypeStruct((N, Dout), jnp.float32),
    )


_deg = _make_deg()
_seg_h = _make_seg(D_H)
_mm_scale = _make_mm_scale(D_H)
_combine_stats = _make_combine_stats(D_H)
_bn_act_mm_h = _make_bn_act_mm(D_H, D_H)
_final = _make_final(D_H, D_Z)


@jax.jit
def kernel(x, edge_index, W1, b1, g1, bt1, W2, b2, g2, bt2,
           W3, b3, g3, bt3, W4, b4):
    src = edge_index[0]
    dst = edge_index[1]
    degp = _deg(dst)

    hn = _mm_scale(x, W1, degp)
    p = _seg_h(hn, src, dst)
    y, st = _combine_stats(p, hn, degp, b1.reshape(1, -1))

    hn = _bn_act_mm_h(y, st, degp, g1.reshape(1, -1), bt1.reshape(1, -1), W2)
    p = _seg_h(hn, src, dst)
    y, st = _combine_stats(p, hn, degp, b2.reshape(1, -1))

    hn = _bn_act_mm_h(y, st, degp, g2.reshape(1, -1), bt2.reshape(1, -1), W3)
    p = _seg_h(hn, src, dst)
    y, st = _combine_stats(p, hn, degp, b3.reshape(1, -1))

    W4p = jnp.pad(W4, ((0, 0), (0, D_H - D_Z)))
    hn = _bn_act_mm_h(y, st, degp, g3.reshape(1, -1), bt3.reshape(1, -1), W4p)
    p = _seg_h(hn, src, dst)
    return _final(p, hn, degp, b4.reshape(1, -1))


# speedup vs baseline: 21.0951x; 2.2001x over previous
import functools

import jax
import jax.numpy as jnp
from jax import lax
from jax.experimental import pallas as pl
from jax.experimental.pallas import tpu as pltpu
from jax.experimental.pallas import tpu_sc as plsc

N = 10000
E = 320000
D_IN = 128
D_H = 128
D_Z = 64

NC, NS = 2, 16
NT = NC * NS
EPT = E // NT
K = 80
CH = EPT // K
N2 = 10240
RPT = N2 // NS
ZR = 128

_MESH = plsc.VectorSubcoreMesh(core_axis_name="c", subcore_axis_name="s")


def _make_seg(D):
    def body(hn, srcs, dsts, out, sidx, didx, rows0, rows1, accum, g0, g1):
        c = lax.axis_index("c")
        s = lax.axis_index("s")
        wid = c * NS + s

        nlanes = D // 16

        def zb(i, _):
            rows0[i // nlanes, pl.ds((i % nlanes) * 16, 16)] = jnp.zeros(
                (16,), jnp.float32)
            return 0

        lax.fori_loop(0, K * nlanes, zb, 0)
        pltpu.sync_copy(srcs.at[pl.ds(wid * EPT, EPT)], sidx)
        pltpu.sync_copy(dsts.at[wid], didx)
        for r in range(RPT // K):
            pltpu.sync_copy(rows0, accum.at[pl.ds(s * RPT + r * K, K)])
        plsc.subcore_barrier()

        def start_g(j, buf, sem):
            pltpu.async_copy(hn.at[sidx.at[pl.ds(j * K, K)]], buf, sem)

        def wait_g(j, buf, sem):
            pltpu.make_async_copy(hn.at[sidx.at[pl.ds(j * K, K)]], buf,
                                  sem).wait()

        def scat(j, buf):
            pltpu.sync_copy(buf, accum.at[didx.at[j]], add=True)

        start_g(0, rows0, g0)
        start_g(1, rows1, g1)

        def ring(jj, _):
            j0 = 2 * jj
            wait_g(j0, rows0, g0)
            scat(j0, rows0)

            @pl.when(j0 + 2 < CH)
            def _():
                start_g(j0 + 2, rows0, g0)

            wait_g(j0 + 1, rows1, g1)
            scat(j0 + 1, rows1)

            @pl.when(j0 + 3 < CH)
            def _():
                start_g(j0 + 3, rows1, g1)

            return 0

        lax.fori_loop(0, CH // 2, ring, 0)
        if CH % 2:
            wait_g(CH - 1, rows0, g0)
            scat(CH - 1, rows0)
        plsc.subcore_barrier()
        pltpu.sync_copy(accum.at[pl.ds(s * RPT, RPT)],
                        out.at[c].at[pl.ds(s * RPT, RPT)])

    return pl.kernel(
        body,
        out_type=jax.ShapeDtypeStruct((NC, N2, D), jnp.float32),
        mesh=_MESH,
        scratch_types=[
            pltpu.VMEM((EPT,), jnp.int32),
            pltpu.VMEM((CH, K), jnp.int32),
            pltpu.VMEM((K, D), jnp.float32),
            pltpu.VMEM((K, D), jnp.float32),
            pltpu.VMEM_SHARED((N2, D), jnp.float32),
            pltpu.SemaphoreType.DMA,
            pltpu.SemaphoreType.DMA,
        ],
    )


def _make_deg():
    DD = 128

    def body(dsts, out, didx, ones, accum, sem):
        c = lax.axis_index("c")
        s = lax.axis_index("s")
        wid = c * NS + s
        nlanes = DD // 16

        def zb(i, _):
            ones[i // nlanes, pl.ds((i % nlanes) * 16, 16)] = jnp.zeros(
                (16,), jnp.float32)
            return 0

        lax.fori_loop(0, K * nlanes, zb, 0)
        pltpu.sync_copy(dsts.at[wid], didx)
        for r in range(RPT // K):
            pltpu.sync_copy(ones, accum.at[pl.ds(s * RPT + r * K, K)])

        def fill_ones(i, _):
            ones[i // nlanes, pl.ds((i % nlanes) * 16, 16)] = jnp.ones(
                (16,), jnp.float32)
            return 0

        lax.fori_loop(0, K * nlanes, fill_ones, 0)
        plsc.subcore_barrier()

        def fire(j, _):
            pltpu.async_copy(ones, accum.at[didx.at[j]], sem, add=True)
            return 0

        lax.fori_loop(0, CH, fire, 0)

        def drain(j, _):
            pltpu.make_async_copy(ones, accum.at[didx.at[j]], sem).wait()
            return 0

        lax.fori_loop(0, CH, drain, 0)
        plsc.subcore_barrier()
        pltpu.sync_copy(accum.at[pl.ds(s * RPT, RPT)],
                        out.at[c].at[pl.ds(s * RPT, RPT)])

    return pl.kernel(
        body,
        out_type=jax.ShapeDtypeStruct((NC, N2, DD), jnp.float32),
        mesh=_MESH,
        scratch_types=[
            pltpu.VMEM((CH, K), jnp.int32),
            pltpu.VMEM((K, DD), jnp.float32),
            pltpu.VMEM_SHARED((N2, DD), jnp.float32),
            pltpu.SemaphoreType.DMA,
        ],
    )


BM = 1000
GR = N // BM


def _dinv_block(degp):
    deg = degp[0, :, 0] + degp[1, :, 0] + 1.0
    return lax.rsqrt(deg)


def _make_mm_scale(Dout):
    def body(x_ref, w_ref, degp_ref, hn_ref):
        dinv = _dinv_block(degp_ref[...])
        hn_ref[...] = dinv[:, None] * jnp.dot(
            x_ref[...], w_ref[...], preferred_element_type=jnp.float32)

    return pl.pallas_call(
        body,
        grid=(GR,),
        in_specs=[
            pl.BlockSpec((BM, D_H), lambda i: (i, 0)),
            pl.BlockSpec((D_H, Dout), lambda i: (0, 0)),
            pl.BlockSpec((2, BM, 128), lambda i: (0, i, 0)),
        ],
        out_specs=pl.BlockSpec((BM, Dout), lambda i: (i, 0)),
        out_shape=jax.ShapeDtypeStruct((N, Dout), jnp.float32),
    )


def _make_combine_stats(D):
    def body(p_ref, hn_ref, degp_ref, b_ref, y_ref, st_ref):
        i = pl.program_id(0)
        dinv = _dinv_block(degp_ref[...])
        y = dinv[:, None] * (p_ref[0] + p_ref[1] + hn_ref[...]) + b_ref[...]
        y_ref[...] = y
        su = jnp.sum(y, axis=0)
        sq = jnp.sum(y * y, axis=0)
        upd = jnp.concatenate(
            [su.reshape(1, D), sq.reshape(1, D),
             jnp.zeros((6, D), jnp.float32)], axis=0)

        @pl.when(i == 0)
        def _():
            st_ref[...] = upd

        @pl.when(i > 0)
        def _():
            st_ref[...] += upd

    return pl.pallas_call(
        body,
        grid=(GR,),
        in_specs=[
            pl.BlockSpec((2, BM, D), lambda i: (0, i, 0)),
            pl.BlockSpec((BM, D), lambda i: (i, 0)),
            pl.BlockSpec((2, BM, 128), lambda i: (0, i, 0)),
            pl.BlockSpec((1, D), lambda i: (0, 0)),
        ],
        out_specs=[
            pl.BlockSpec((BM, D), lambda i: (i, 0)),
            pl.BlockSpec((8, D), lambda i: (0, 0)),
        ],
        out_shape=[
            jax.ShapeDtypeStruct((N, D), jnp.float32),
            jax.ShapeDtypeStruct((8, D), jnp.float32),
        ],
    )


def _make_bn_act_mm(D, Dout):
    def body(y_ref, st_ref, degp_ref, g_ref, bt_ref, w_ref, hn_ref):
        dinv = _dinv_block(degp_ref[...])
        st = st_ref[...]
        m = st[0:1, :] / N
        v = st[1:2, :] / N - m * m
        a = (y_ref[...] - m) * lax.rsqrt(v + 1e-5) * g_ref[...] + bt_ref[...]
        a = jnp.where(a >= 0, a, 0.01 * a)
        hn_ref[...] = dinv[:, None] * jnp.dot(
            a, w_ref[...], preferred_element_type=jnp.float32)

    return pl.pallas_call(
        body,
        grid=(GR,),
        in_specs=[
            pl.BlockSpec((BM, D), lambda i: (i, 0)),
            pl.BlockSpec((8, D), lambda i: (0, 0)),
            pl.BlockSpec((2, BM, 128), lambda i: (0, i, 0)),
            pl.BlockSpec((1, D), lambda i: (0, 0)),
            pl.BlockSpec((1, D), lambda i: (0, 0)),
            pl.BlockSpec((D, Dout), lambda i: (0, 0)),
        ],
        out_specs=pl.BlockSpec((BM, Dout), lambda i: (i, 0)),
        out_shape=jax.ShapeDtypeStruct((N, Dout), jnp.float32),
    )


def _make_final(D, Dout):
    def body(p_ref, hn_ref, degp_ref, b_ref, out_ref):
        dinv = _dinv_block(degp_ref[...])
        full = dinv[:, None] * (p_ref[0] + p_ref[1] + hn_ref[...])
        out_ref[...] = full[:, :Dout] + b_ref[...]

    return pl.pallas_call(
        body,
        grid=(GR,),
        in_specs=[
            pl.BlockSpec((2, BM, D), lambda i: (0, i, 0)),
            pl.BlockSpec((BM, D), lambda i: (i, 0)),
            pl.BlockSpec((2, BM, 128), lambda i: (0, i, 0)),
            pl.BlockSpec((1, Dout), lambda i: (0, 0)),
        ],
        out_specs=pl.BlockSpec((BM, Dout), lambda i: (i, 0)),
        out_shape=jax.ShapeDtypeStruct((N, Dout), jnp.float32),
    )


_deg = _make_deg()
_seg_h = _make_seg(D_H)
_mm_scale = _make_mm_scale(D_H)
_combine_stats = _make_combine_stats(D_H)
_bn_act_mm_h = _make_bn_act_mm(D_H, D_H)
_final = _make_final(D_H, D_Z)


@jax.jit
def kernel(x, edge_index, W1, b1, g1, bt1, W2, b2, g2, bt2,
           W3, b3, g3, bt3, W4, b4):
    src = edge_index[0]
    dst = edge_index[1].reshape(NT, CH, K)
    degp = _deg(dst)

    hn = _mm_scale(x, W1, degp)
    p = _seg_h(hn, src, dst)
    y, st = _combine_stats(p, hn, degp, b1.reshape(1, -1))

    hn = _bn_act_mm_h(y, st, degp, g1.reshape(1, -1), bt1.reshape(1, -1), W2)
    p = _seg_h(hn, src, dst)
    y, st = _combine_stats(p, hn, degp, b2.reshape(1, -1))

    hn = _bn_act_mm_h(y, st, degp, g2.reshape(1, -1), bt2.reshape(1, -1), W3)
    p = _seg_h(hn, src, dst)
    y, st = _combine_stats(p, hn, degp, b3.reshape(1, -1))

    W4p = jnp.pad(W4, ((0, 0), (0, D_H - D_Z)))
    hn = _bn_act_mm_h(y, st, degp, g3.reshape(1, -1), bt3.reshape(1, -1), W4p)
    p = _seg_h(hn, src, dst)
    return _final(p, hn, degp, b4.reshape(1, -1))
